# Initial kernel scaffold; baseline (speedup 1.0000x reference)
import jax
import jax.numpy as jnp
from jax.experimental import pallas as pl


def kernel(states, x_bits, adj_list, adj_mask, lut, input_nodes, W, b):
    raise NotImplementedError("write your pallas kernel here")


# speedup vs baseline: 6.7418x; 6.7418x over previous
import functools

import numpy as np
import jax
import jax.numpy as jnp
from jax import lax
from jax.experimental import pallas as pl
from jax.experimental.pallas import tpu as pltpu
from jax.experimental.pallas import tpu_sc as plsc

N = 100000
K = 8
B = 64
NI = 128
BPF = 8
NXQ = NI * BPF

NBLK = 2048
GRID = 49
NPAD = NBLK * GRID
CHUNK = NPAD // 16
SUB = 224
NSUB = CHUNK // SUB
NGRP = SUB // 16

_PACK_M = np.zeros((256, 32), np.float32)
for _v in range(256):
    _PACK_M[_v, _v >> 3] = float(1 << (_v & 7))

def _i32(v):
    v &= 0xFFFFFFFF
    return v - (1 << 32) if v >= (1 << 31) else v

_RMASK = [_i32(0x01010101 << r) for r in range(8)]


def _pack_body(states_ref, lut_ref, xb_ref, m_ref, pstates_ref, plut_ref,
               xw_ref):
    s = states_ref[...]
    bi = lax.broadcasted_iota(jnp.int32, (B, NBLK), 0)
    c = s << (bi & 31)
    w0 = jnp.sum(jnp.where(bi < 32, c, 0), axis=0, keepdims=True)
    w1 = jnp.sum(jnp.where(bi >= 32, c, 0), axis=0, keepdims=True)
    pstates_ref[...] = jnp.concatenate([w0, w1], axis=0)

    lf = lut_ref[...].astype(jnp.float32)
    pb = lax.dot_general(lf, m_ref[...], (((1,), (0,)), ((), ())),
                         preferred_element_type=jnp.float32)
    plut_ref[...] = pb.astype(jnp.int32)

    xb = xb_ref[...]
    xi = lax.broadcasted_iota(jnp.int32, (B, NXQ), 0)
    xc = xb << (xi & 31)
    x0 = jnp.sum(jnp.where(xi < 32, xc, 0), axis=0, keepdims=True)
    x1 = jnp.sum(jnp.where(xi >= 32, xc, 0), axis=0, keepdims=True)
    xw_ref[...] = jnp.concatenate([x0, x1], axis=0)


_pack = pl.pallas_call(
    _pack_body,
    grid=(GRID,),
    in_specs=[
        pl.BlockSpec((B, NBLK), lambda i: (0, i)),
        pl.BlockSpec((NBLK, 256), lambda i: (i, 0)),
        pl.BlockSpec((B, NXQ), lambda i: (0, 0)),
        pl.BlockSpec((256, 32), lambda i: (0, 0)),
    ],
    out_specs=[
        pl.BlockSpec((2, NBLK), lambda i: (0, i)),
        pl.BlockSpec((NBLK, 32), lambda i: (i, 0)),
        pl.BlockSpec((2, NXQ), lambda i: (0, 0)),
    ],
    out_shape=[
        jax.ShapeDtypeStruct((2, NPAD), jnp.int32),
        jax.ShapeDtypeStruct((NPAD, 32), jnp.int32),
        jax.ShapeDtypeStruct((2, NXQ), jnp.int32),
    ],
)


_SC_MESH = plsc.VectorSubcoreMesh(core_axis_name="c", subcore_axis_name="s")


@functools.partial(
    pl.kernel,
    mesh=_SC_MESH,
    out_type=jax.ShapeDtypeStruct((2 * NPAD,), jnp.int32),
    compiler_params=pltpu.CompilerParams(needs_layout_passes=False),
    scratch_types=[
        pltpu.VMEM((NPAD,), jnp.int32),
        pltpu.VMEM((SUB * 8,), jnp.int32),
        pltpu.VMEM((SUB * 32,), jnp.int32),
        pltpu.VMEM((CHUNK,), jnp.int32),
        pltpu.VMEM((NXQ,), jnp.int32),
        pltpu.VMEM((NXQ,), jnp.int32),
    ],
)
def _sc_step(ps_hbm, adj_hbm, plut_hbm, nodes_hbm, xw_hbm, out_hbm,
             table_v, adj_v, plut_v, out_v, nodes_v, xw_v):
    w = lax.axis_index("c")
    cid = lax.axis_index("s")
    pltpu.sync_copy(ps_hbm.at[pl.ds(w * NPAD, NPAD)], table_v)
    pltpu.sync_copy(nodes_hbm, nodes_v)
    pltpu.sync_copy(xw_hbm.at[pl.ds(w * NXQ, NXQ)], xw_v)

    it = lax.iota(jnp.int32, 16)

    def xor_body(q, carry):
        nv = nodes_v[pl.ds(q * 16, 16)]
        xv = xw_v[pl.ds(q * 16, 16)]
        old = plsc.load_gather(table_v, [nv])
        plsc.store_scatter(table_v, [nv], old ^ xv)
        return carry

    lax.fori_loop(0, NXQ // 16, xor_body, 0)

    node0 = cid * CHUNK

    def sub_body(sidx, carry):
        pltpu.sync_copy(
            adj_hbm.at[pl.ds((node0 + sidx * SUB) * 8, SUB * 8)], adj_v)
        pltpu.sync_copy(
            plut_hbm.at[pl.ds((node0 + sidx * SUB) * 32, SUB * 32)], plut_v)

        def grp_body(g, carry2):
            nl = g * 16 + it
            nl8 = nl * 8
            gj = []
            for j in range(8):
                aj = plsc.load_gather(adj_v, [nl8 + j])
                gj.append(plsc.load_gather(table_v, [aj]))
            outs = []
            for r in range(8):
                acc = jnp.zeros((16,), jnp.int32)
                for j in range(8):
                    t = gj[j] & _RMASK[r]
                    d = 7 - j - r
                    if d > 0:
                        t = t << d
                    elif d < 0:
                        t = lax.shift_right_logical(t, -d)
                    acc = acc | t
                outs.append(acc)
            nl32 = nl * 32
            res = jnp.zeros((16,), jnp.int32)
            for r in range(8):
                for kq in range(4):
                    idxv = lax.shift_right_logical(outs[r], 8 * kq) & 0xFF
                    bad = nl32 + lax.shift_right_logical(idxv, 3)
                    bytev = plsc.load_gather(plut_v, [bad])
                    bit = lax.shift_right_logical(bytev, idxv & 7) & 1
                    res = res | (bit << (r + 8 * kq))
            out_v[pl.ds(sidx * SUB + g * 16, 16)] = res
            return carry2

        lax.fori_loop(0, NGRP, grp_body, 0)
        return carry

    lax.fori_loop(0, NSUB, sub_body, 0)
    pltpu.sync_copy(out_v, out_hbm.at[pl.ds(w * NPAD + node0, CHUNK)])


def _readout_body(pnew_ref, w_ref, b_ref, out_ref):
    i = pl.program_id(0)
    pw = pnew_ref[...]
    bi = lax.broadcasted_iota(jnp.int32, (B, NBLK), 0)
    wsel = jnp.where(bi < 32, pw[0:1, :], pw[1:2, :])
    bits = (lax.shift_right_logical(wsel, bi & 31) & 1).astype(jnp.float32)
    ni = lax.broadcasted_iota(jnp.int32, (NI, NBLK), 1) + i * NBLK
    wv = jnp.where(ni < N, w_ref[...], 0.0)
    contrib = lax.dot_general(bits, wv, (((1,), (1,)), ((), ())),
                              preferred_element_type=jnp.float32)

    @pl.when(i == 0)
    def _():
        out_ref[...] = jnp.broadcast_to(b_ref[...], (B, NI))

    out_ref[...] += contrib


_readout = pl.pallas_call(
    _readout_body,
    grid=(GRID,),
    in_specs=[
        pl.BlockSpec((2, NBLK), lambda i: (0, i)),
        pl.BlockSpec((NI, NBLK), lambda i: (0, i)),
        pl.BlockSpec((1, NI), lambda i: (0, 0)),
    ],
    out_specs=pl.BlockSpec((B, NI), lambda i: (0, 0)),
    out_shape=jax.ShapeDtypeStruct((B, NI), jnp.float32),
)


def kernel(states, x_bits, adj_list, adj_mask, lut, input_nodes, W, b):
    del adj_mask
    xb = x_bits.reshape(B, NXQ).astype(jnp.int32)
    pstates, plut, xw = _pack(states.astype(jnp.int32),
                              lut.astype(jnp.int32), xb,
                              jnp.asarray(_PACK_M))
    adj_flat = jnp.pad(adj_list.astype(jnp.int32),
                       ((0, NPAD - N), (0, 0))).reshape(-1)
    pnew_flat = _sc_step(pstates.reshape(-1), adj_flat, plut.reshape(-1),
                         input_nodes.reshape(-1).astype(jnp.int32),
                         xw.reshape(-1))
    pnew = pnew_flat.reshape(2, NPAD)
    out = _readout(pnew, W, b.reshape(1, NI))
    return out


# speedup vs baseline: 11.0902x; 1.6450x over previous
import functools

import numpy as np
import jax
import jax.numpy as jnp
from jax import lax
from jax.experimental import pallas as pl
from jax.experimental.pallas import tpu as pltpu
from jax.experimental.pallas import tpu_sc as plsc

N = 100000
K = 8
B = 64
NI = 128
BPF = 8
NXQ = NI * BPF

NBLK = 2560
GRID = 40
NPAD = NBLK * GRID
CHUNK = NPAD // 16
SUB = 256
NSUB = CHUNK // SUB
NGRP = SUB // 16

def _rev8(v):
    r = 0
    for i in range(8):
        r |= ((v >> i) & 1) << (7 - i)
    return r

_PACK_MT = np.zeros((32, 256), np.float32)
for _v in range(256):
    _vp = _rev8(_v)
    _PACK_MT[_vp >> 3, _v] = float(1 << (_vp & 7))

_BFLY = {1: 0x55555555, 2: 0x33333333, 4: 0x0F0F0F0F}


def _i32(v):
    v &= 0xFFFFFFFF
    return v - (1 << 32) if v >= (1 << 31) else v


def _pack_body(states_ref, lut_ref, xb_ref, m_ref, adjt_ref,
               pstates_ref, plut_ref, xw_ref, adjc_ref):
    s = states_ref[...]
    bi = lax.broadcasted_iota(jnp.int32, (B, NBLK), 0)
    c = s << (bi & 31)
    w0 = jnp.sum(jnp.where(bi < 32, c, 0), axis=0, keepdims=True)
    w1 = jnp.sum(jnp.where(bi >= 32, c, 0), axis=0, keepdims=True)
    pstates_ref[...] = jnp.concatenate([w0, w1], axis=0)

    lf = lut_ref[...].astype(jnp.float32)
    pb = lax.dot_general(m_ref[...], lf, (((1,), (1,)), ((), ())),
                         preferred_element_type=jnp.float32)
    plut_ref[...] = pb.astype(jnp.int32)

    xb = xb_ref[...]
    xi = lax.broadcasted_iota(jnp.int32, (B, NXQ), 0)
    xc = xb << (xi & 31)
    x0 = jnp.sum(jnp.where(xi < 32, xc, 0), axis=0, keepdims=True)
    x1 = jnp.sum(jnp.where(xi >= 32, xc, 0), axis=0, keepdims=True)
    xw_ref[...] = jnp.concatenate([x0, x1], axis=0)

    a = adjt_ref[...]
    adjc_ref[...] = jnp.minimum(jnp.maximum(a, 0), N - 1)


_pack = pl.pallas_call(
    _pack_body,
    grid=(GRID,),
    in_specs=[
        pl.BlockSpec((B, NBLK), lambda i: (0, i)),
        pl.BlockSpec((NBLK, 256), lambda i: (i, 0)),
        pl.BlockSpec((B, NXQ), lambda i: (0, 0)),
        pl.BlockSpec((32, 256), lambda i: (0, 0)),
        pl.BlockSpec((K, NBLK), lambda i: (0, i)),
    ],
    out_specs=[
        pl.BlockSpec((2, NBLK), lambda i: (0, i)),
        pl.BlockSpec((32, NBLK), lambda i: (0, i)),
        pl.BlockSpec((2, NXQ), lambda i: (0, 0)),
        pl.BlockSpec((K, NBLK), lambda i: (0, i)),
    ],
    out_shape=[
        jax.ShapeDtypeStruct((2, NPAD), jnp.int32),
        jax.ShapeDtypeStruct((32, NPAD), jnp.int32),
        jax.ShapeDtypeStruct((2, NXQ), jnp.int32),
        jax.ShapeDtypeStruct((K, NPAD), jnp.int32),
    ],
)


_SC_MESH = plsc.VectorSubcoreMesh(core_axis_name="c", subcore_axis_name="s")


@functools.partial(
    pl.kernel,
    mesh=_SC_MESH,
    out_type=jax.ShapeDtypeStruct((2 * NPAD,), jnp.int32),
    compiler_params=pltpu.CompilerParams(needs_layout_passes=False),
    scratch_types=[
        pltpu.VMEM((NPAD,), jnp.int32),
        pltpu.VMEM((K, SUB), jnp.int32),
        pltpu.VMEM((32, SUB), jnp.int32),
        pltpu.VMEM((CHUNK,), jnp.int32),
        pltpu.VMEM((NXQ,), jnp.int32),
        pltpu.VMEM((NXQ,), jnp.int32),
    ],
)
def _sc_step(ps_hbm, adj_hbm, plut_hbm, nodes_hbm, xw_hbm, out_hbm,
             table_v, adj_v, plut_v, out_v, nodes_v, xw_v):
    w = lax.axis_index("c")
    cid = lax.axis_index("s")
    pltpu.sync_copy(ps_hbm.at[pl.ds(w * NPAD, NPAD)], table_v)
    pltpu.sync_copy(nodes_hbm, nodes_v)
    pltpu.sync_copy(xw_hbm.at[pl.ds(w * NXQ, NXQ)], xw_v)

    it = lax.iota(jnp.int32, 16)

    def xor_body(q, carry):
        nv = nodes_v[pl.ds(q * 16, 16)]
        xv = xw_v[pl.ds(q * 16, 16)]
        old = plsc.load_gather(table_v, [nv])
        plsc.store_scatter(table_v, [nv], old ^ xv)
        return carry

    lax.fori_loop(0, NXQ // 16, xor_body, 0)

    node0 = cid * CHUNK

    def sub_body(sidx, carry):
        base = node0 + sidx * SUB
        pltpu.sync_copy(adj_hbm.at[:, pl.ds(base, SUB)], adj_v)
        pltpu.sync_copy(plut_hbm.at[:, pl.ds(base, SUB)], plut_v)

        def grp_body(g, carry2):
            nl = g * 16 + it
            gj = []
            for j in range(8):
                aj = plsc.load_gather(
                    adj_v, [jnp.full((16,), j, jnp.int32), nl])
                gj.append(plsc.load_gather(table_v, [aj]))
            for d in (4, 2, 1):
                m = _i32(_BFLY[d])
                for i in range(8):
                    if i & d:
                        continue
                    t = (lax.shift_right_logical(gj[i], d) ^ gj[i + d]) & m
                    gj[i + d] = gj[i + d] ^ t
                    gj[i] = gj[i] ^ (t << d)
            res = jnp.zeros((16,), jnp.int32)
            for r in range(8):
                o = gj[r]
                for kq in range(4):
                    if kq < 3:
                        idxv = o & 0xFF
                        o = lax.shift_right_logical(o, 8)
                    else:
                        idxv = o
                    bad = lax.shift_right_logical(idxv, 3)
                    bytev = plsc.load_gather(plut_v, [bad, nl])
                    bit = lax.shift_right_logical(bytev, idxv & 7) & 1
                    res = res | (bit << (r + 8 * kq))
            out_v[pl.ds(sidx * SUB + g * 16, 16)] = res
            return carry2

        lax.fori_loop(0, NGRP, grp_body, 0)
        return carry

    lax.fori_loop(0, NSUB, sub_body, 0)
    pltpu.sync_copy(out_v, out_hbm.at[pl.ds(w * NPAD + node0, CHUNK)])


def _readout_body(pnew_ref, w_ref, b_ref, out_ref):
    i = pl.program_id(0)
    pw = pnew_ref[...]
    bi = lax.broadcasted_iota(jnp.int32, (B, NBLK), 0)
    wsel = jnp.where(bi < 32, pw[0:1, :], pw[1:2, :])
    bits = (lax.shift_right_logical(wsel, bi & 31) & 1).astype(jnp.float32)
    ni = lax.broadcasted_iota(jnp.int32, (NI, NBLK), 1) + i * NBLK
    wv = jnp.where(ni < N, w_ref[...], 0.0)
    contrib = lax.dot_general(bits, wv, (((1,), (1,)), ((), ())),
                              preferred_element_type=jnp.float32)

    @pl.when(i == 0)
    def _():
        out_ref[...] = jnp.broadcast_to(b_ref[...], (B, NI))

    out_ref[...] += contrib


_readout = pl.pallas_call(
    _readout_body,
    grid=(GRID,),
    in_specs=[
        pl.BlockSpec((2, NBLK), lambda i: (0, i)),
        pl.BlockSpec((NI, NBLK), lambda i: (0, i)),
        pl.BlockSpec((1, NI), lambda i: (0, 0)),
    ],
    out_specs=pl.BlockSpec((B, NI), lambda i: (0, 0)),
    out_shape=jax.ShapeDtypeStruct((B, NI), jnp.float32),
)


def kernel(states, x_bits, adj_list, adj_mask, lut, input_nodes, W, b):
    del adj_mask
    xb = x_bits.reshape(B, NXQ).astype(jnp.int32)
    pstates, plut, xw, adjc = _pack(states.astype(jnp.int32),
                                    lut.astype(jnp.int32), xb,
                                    jnp.asarray(_PACK_MT),
                                    adj_list.astype(jnp.int32).T)
    pnew_flat = _sc_step(pstates.reshape(-1), adjc, plut,
                         input_nodes.reshape(-1).astype(jnp.int32),
                         xw.reshape(-1))
    out = _readout(pnew_flat.reshape(2, NPAD), W, b.reshape(1, NI))
    return out


# speedup vs baseline: 14.5731x; 1.3141x over previous
import functools

import numpy as np
import jax
import jax.numpy as jnp
from jax import lax
from jax.experimental import pallas as pl
from jax.experimental.pallas import tpu as pltpu
from jax.experimental.pallas import tpu_sc as plsc

N = 100000
K = 8
B = 64
NI = 128
BPF = 8
NXQ = NI * BPF

NBLK = 5120
GRID = 20
NPAD = NBLK * GRID
CHUNK = NPAD // 16
SUB = 128
NSUB = CHUNK // SUB
NGRP = SUB // 16

def _rev8(v):
    r = 0
    for i in range(8):
        r |= ((v >> i) & 1) << (7 - i)
    return r

_PACK_MT = np.zeros((32, 256), np.float32)
for _v in range(256):
    _vp = _rev8(_v)
    _PACK_MT[_vp >> 3, _v] = float(1 << (_vp & 7))

_BFLY = {1: 0x55555555, 2: 0x33333333, 4: 0x0F0F0F0F}


def _i32(v):
    v &= 0xFFFFFFFF
    return v - (1 << 32) if v >= (1 << 31) else v


def _pack_body(states_ref, lut_ref, xb_ref, m_ref, adjt_ref,
               pstates_ref, plut_ref, xw_ref, adjc_ref):
    s = states_ref[...]
    bi = lax.broadcasted_iota(jnp.int32, (B, NBLK), 0)
    c = s << (bi & 31)
    w0 = jnp.sum(jnp.where(bi < 32, c, 0), axis=0, keepdims=True)
    w1 = jnp.sum(jnp.where(bi >= 32, c, 0), axis=0, keepdims=True)
    pstates_ref[...] = jnp.concatenate([w0, w1], axis=0)

    lf = lut_ref[...].astype(jnp.float32)
    pb = lax.dot_general(m_ref[...], lf, (((1,), (1,)), ((), ())),
                         preferred_element_type=jnp.float32)
    plut_ref[...] = pb.astype(jnp.int32)

    xb = xb_ref[...]
    xi = lax.broadcasted_iota(jnp.int32, (B, NXQ), 0)
    xc = xb << (xi & 31)
    x0 = jnp.sum(jnp.where(xi < 32, xc, 0), axis=0, keepdims=True)
    x1 = jnp.sum(jnp.where(xi >= 32, xc, 0), axis=0, keepdims=True)
    xw_ref[...] = jnp.concatenate([x0, x1], axis=0)

    a = adjt_ref[...]
    adjc_ref[...] = jnp.minimum(jnp.maximum(a, 0), N - 1)


_pack = pl.pallas_call(
    _pack_body,
    grid=(GRID,),
    in_specs=[
        pl.BlockSpec((B, NBLK), lambda i: (0, i)),
        pl.BlockSpec((NBLK, 256), lambda i: (i, 0)),
        pl.BlockSpec((B, NXQ), lambda i: (0, 0)),
        pl.BlockSpec((32, 256), lambda i: (0, 0)),
        pl.BlockSpec((K, NBLK), lambda i: (0, i)),
    ],
    out_specs=[
        pl.BlockSpec((2, NBLK), lambda i: (0, i)),
        pl.BlockSpec((32, NBLK), lambda i: (0, i)),
        pl.BlockSpec((2, NXQ), lambda i: (0, 0)),
        pl.BlockSpec((K, NBLK), lambda i: (0, i)),
    ],
    out_shape=[
        jax.ShapeDtypeStruct((2, NPAD), jnp.int32),
        jax.ShapeDtypeStruct((32, NPAD), jnp.int32),
        jax.ShapeDtypeStruct((2, NXQ), jnp.int32),
        jax.ShapeDtypeStruct((K, NPAD), jnp.int32),
    ],
)


_SC_MESH = plsc.VectorSubcoreMesh(core_axis_name="c", subcore_axis_name="s")


@functools.partial(
    pl.kernel,
    mesh=_SC_MESH,
    out_type=jax.ShapeDtypeStruct((2 * NPAD,), jnp.int32),
    compiler_params=pltpu.CompilerParams(needs_layout_passes=False),
    scratch_types=[
        pltpu.VMEM((NPAD,), jnp.int32),
        pltpu.VMEM((K, SUB), jnp.int32),
        pltpu.VMEM((K, SUB), jnp.int32),
        pltpu.VMEM((32, SUB), jnp.int32),
        pltpu.VMEM((32, SUB), jnp.int32),
        pltpu.VMEM((CHUNK,), jnp.int32),
        pltpu.VMEM((NXQ,), jnp.int32),
        pltpu.VMEM((NXQ,), jnp.int32),
        pltpu.SemaphoreType.DMA,
        pltpu.SemaphoreType.DMA,
    ],
)
def _sc_step(ps_hbm, adj_hbm, plut_hbm, nodes_hbm, xw_hbm, out_hbm,
             table_v, adj_v0, adj_v1, plut_v0, plut_v1, out_v, nodes_v,
             xw_v, sem0, sem1):
    w = lax.axis_index("c")
    cid = lax.axis_index("s")
    pltpu.sync_copy(ps_hbm.at[pl.ds(w * NPAD, NPAD)], table_v)
    pltpu.sync_copy(nodes_hbm, nodes_v)
    pltpu.sync_copy(xw_hbm.at[pl.ds(w * NXQ, NXQ)], xw_v)

    it = lax.iota(jnp.int32, 16)

    def xor_body(q, carry):
        nv = nodes_v[pl.ds(q * 16, 16)]
        xv = xw_v[pl.ds(q * 16, 16)]
        old = plsc.load_gather(table_v, [nv])
        plsc.store_scatter(table_v, [nv], old ^ xv)
        return carry

    lax.fori_loop(0, NXQ // 16, xor_body, 0)

    node0 = cid * CHUNK

    def fetch(sidx, adj_vx, plut_vx, semx):
        base = node0 + sidx * SUB
        pltpu.async_copy(adj_hbm.at[:, pl.ds(base, SUB)], adj_vx, semx)
        pltpu.async_copy(plut_hbm.at[:, pl.ds(base, SUB)], plut_vx, semx)

    def drain(adj_vx, plut_vx, semx):
        pltpu.make_async_copy(
            adj_hbm.at[:, pl.ds(node0, SUB)], adj_vx, semx).wait()
        pltpu.make_async_copy(
            plut_hbm.at[:, pl.ds(node0, SUB)], plut_vx, semx).wait()

    def compute(sidx, adj_vx, plut_vx):
        def grp_body(g, carry2):
            nl = g * 16 + it
            g16 = g * 16
            gj = []
            for j in range(8):
                aj = adj_vx[j, pl.ds(g16, 16)]
                gj.append(plsc.load_gather(table_v, [aj]))
            for d in (4, 2, 1):
                m = _i32(_BFLY[d])
                for i in range(8):
                    if i & d:
                        continue
                    t = (lax.shift_right_logical(gj[i], d) ^ gj[i + d]) & m
                    gj[i + d] = gj[i + d] ^ t
                    gj[i] = gj[i] ^ (t << d)
            acc = [jnp.zeros((16,), jnp.int32) for _ in range(4)]
            for r in range(8):
                o = gj[r]
                for kq in range(4):
                    if kq < 3:
                        idxv = o & 0xFF
                        o = lax.shift_right_logical(o, 8)
                    else:
                        idxv = o
                    bad = lax.shift_right_logical(idxv, 3)
                    bytev = plsc.load_gather(plut_vx, [bad, nl])
                    bit = lax.shift_right_logical(bytev, idxv & 7) & 1
                    acc[kq] = acc[kq] | (bit << (r + 8 * kq))
            res = (acc[0] | acc[1]) | (acc[2] | acc[3])
            out_v[pl.ds(sidx * SUB + g16, 16)] = res
            return carry2

        lax.fori_loop(0, NGRP, grp_body, 0)

    fetch(0, adj_v0, plut_v0, sem0)

    def pair_body(t, carry):
        s0 = 2 * t
        drain(adj_v0, plut_v0, sem0)
        fetch(s0 + 1, adj_v1, plut_v1, sem1)
        compute(s0, adj_v0, plut_v0)
        drain(adj_v1, plut_v1, sem1)

        @pl.when(s0 + 2 < NSUB)
        def _():
            fetch(s0 + 2, adj_v0, plut_v0, sem0)

        compute(s0 + 1, adj_v1, plut_v1)
        return carry

    lax.fori_loop(0, NSUB // 2, pair_body, 0)
    pltpu.sync_copy(out_v, out_hbm.at[pl.ds(w * NPAD + node0, CHUNK)])


def _readout_body(pnew_ref, w_ref, b_ref, out_ref):
    i = pl.program_id(0)
    pw = pnew_ref[...]
    bi = lax.broadcasted_iota(jnp.int32, (B, NBLK), 0)
    wsel = jnp.where(bi < 32, pw[0:1, :], pw[1:2, :])
    bits = (lax.shift_right_logical(wsel, bi & 31) & 1).astype(jnp.float32)
    ni = lax.broadcasted_iota(jnp.int32, (NI, NBLK), 1) + i * NBLK
    wv = jnp.where(ni < N, w_ref[...], 0.0)
    contrib = lax.dot_general(bits, wv, (((1,), (1,)), ((), ())),
                              preferred_element_type=jnp.float32)

    @pl.when(i == 0)
    def _():
        out_ref[...] = jnp.broadcast_to(b_ref[...], (B, NI))

    out_ref[...] += contrib


_readout = pl.pallas_call(
    _readout_body,
    grid=(GRID,),
    in_specs=[
        pl.BlockSpec((2, NBLK), lambda i: (0, i)),
        pl.BlockSpec((NI, NBLK), lambda i: (0, i)),
        pl.BlockSpec((1, NI), lambda i: (0, 0)),
    ],
    out_specs=pl.BlockSpec((B, NI), lambda i: (0, 0)),
    out_shape=jax.ShapeDtypeStruct((B, NI), jnp.float32),
)


def kernel(states, x_bits, adj_list, adj_mask, lut, input_nodes, W, b):
    del adj_mask
    xb = x_bits.reshape(B, NXQ).astype(jnp.int32)
    pstates, plut, xw, adjc = _pack(states.astype(jnp.int32),
                                    lut.astype(jnp.int32), xb,
                                    jnp.asarray(_PACK_MT),
                                    adj_list.astype(jnp.int32).T)
    pnew_flat = _sc_step(pstates.reshape(-1), adjc, plut,
                         input_nodes.reshape(-1).astype(jnp.int32),
                         xw.reshape(-1))
    out = _readout(pnew_flat.reshape(2, NPAD), W, b.reshape(1, NI))
    return out
